# Initial kernel scaffold; baseline (speedup 1.0000x reference)
import jax
import jax.numpy as jnp
from jax.experimental import pallas as pl


def kernel(feats, edge_index, W1, b1, W2, b2, eps):
    raise NotImplementedError("write your pallas kernel here")


# speedup vs baseline: 4.3863x; 4.3863x over previous
import functools

import jax
import jax.numpy as jnp
from jax import lax
from jax.experimental import pallas as pl
from jax.experimental.pallas import tpu as pltpu
from jax.experimental.pallas import tpu_sc as plsc

N_NODES = 10000
N_EDGES = 320000
D = 128
NUM_LAYERS = 3

NC = 2
NS = 16
CHUNK = 128
CPW = 79
EPW = CPW * CHUNK
E_PAD = NC * NS * EPW
N_PAD = 10240
ZROWS = N_PAD // NS
ZCH = ZROWS // CHUNK


def _sc_agg_body(src_hbm, dst_hbm, h_hbm, out_hbm, srcv, dstv, rowsv, agg_sh, sem):
    c = lax.axis_index("c")
    s = lax.axis_index("s")
    w = c * NS + s

    pltpu.sync_copy(src_hbm.at[w], srcv)
    pltpu.sync_copy(dst_hbm.at[w], dstv)

    def _zero(k, _):
        i = k // (D // 16)
        j = k % (D // 16)
        rowsv[i, pl.ds(j * 16, 16)] = jnp.zeros((16,), jnp.float32)
        return 0

    lax.fori_loop(0, CHUNK * (D // 16), _zero, 0)
    for z in range(ZCH):
        pltpu.sync_copy(rowsv, agg_sh.at[pl.ds(s * ZROWS + z * CHUNK, CHUNK)])
    plsc.subcore_barrier()

    def _chunk(j, _):
        pltpu.async_copy(h_hbm.at[srcv.at[j]], rowsv, sem).wait()
        pltpu.sync_copy(rowsv, agg_sh.at[dstv.at[j]], add=True)
        return 0

    lax.fori_loop(0, CPW, _chunk, 0)
    plsc.subcore_barrier()

    for z in range(ZCH):
        r0 = s * ZROWS + z * CHUNK
        pltpu.sync_copy(agg_sh.at[pl.ds(r0, CHUNK)], rowsv)
        pltpu.sync_copy(rowsv, out_hbm.at[c].at[pl.ds(r0, CHUNK)])


_sc_agg = pl.kernel(
    _sc_agg_body,
    out_type=jax.ShapeDtypeStruct((NC, N_PAD, D), jnp.float32),
    mesh=plsc.VectorSubcoreMesh(
        core_axis_name="c", subcore_axis_name="s", num_cores=NC, num_subcores=NS
    ),
    scratch_types=[
        pltpu.VMEM((CPW, CHUNK), jnp.int32),
        pltpu.VMEM((CPW, CHUNK), jnp.int32),
        pltpu.VMEM((CHUNK, D), jnp.float32),
        pltpu.VMEM_SHARED((N_PAD, D), jnp.float32),
        pltpu.SemaphoreType.DMA,
    ],
)


def _mlp_body(eps_ref, h_ref, a0_ref, a1_ref, w1_ref, b1_ref, w2_ref, b2_ref, o_ref):
    rst = h_ref[...] * (1.0 + eps_ref[0, 0]) + a0_ref[...] + a1_ref[...]
    hid = jnp.maximum(
        jnp.dot(rst, w1_ref[...], preferred_element_type=jnp.float32) + b1_ref[...], 0.0
    )
    o_ref[...] = jnp.dot(hid, w2_ref[...], preferred_element_type=jnp.float32) + b2_ref[...]


def _mlp_sum_body(eps_ref, h_ref, a0_ref, a1_ref, w1_ref, b1_ref, w2_ref, b2_ref, o_ref):
    rst = h_ref[...] * (1.0 + eps_ref[0, 0]) + a0_ref[...] + a1_ref[...]
    hid = jnp.maximum(
        jnp.dot(rst, w1_ref[...], preferred_element_type=jnp.float32) + b1_ref[...], 0.0
    )
    out = jnp.dot(hid, w2_ref[...], preferred_element_type=jnp.float32) + b2_ref[...]

    @pl.when(pl.program_id(0) == 0)
    def _():
        o_ref[...] = jnp.zeros_like(o_ref)

    o_ref[...] += jnp.sum(out, axis=0, keepdims=True)


_MLP_BLOCK = 1000
_MLP_GRID = N_NODES // _MLP_BLOCK


def _mlp_call(body, out_shape, out_spec):
    return pl.pallas_call(
        body,
        grid=(_MLP_GRID,),
        in_specs=[
            pl.BlockSpec(memory_space=pltpu.SMEM),
            pl.BlockSpec((_MLP_BLOCK, D), lambda i: (i, 0)),
            pl.BlockSpec((_MLP_BLOCK, D), lambda i: (i, 0)),
            pl.BlockSpec((_MLP_BLOCK, D), lambda i: (i, 0)),
            pl.BlockSpec((D, D), lambda i: (0, 0)),
            pl.BlockSpec((1, D), lambda i: (0, 0)),
            pl.BlockSpec((D, D), lambda i: (0, 0)),
            pl.BlockSpec((1, D), lambda i: (0, 0)),
        ],
        out_specs=out_spec,
        out_shape=out_shape,
    )


_mlp = _mlp_call(
    _mlp_body,
    jax.ShapeDtypeStruct((N_NODES, D), jnp.float32),
    pl.BlockSpec((_MLP_BLOCK, D), lambda i: (i, 0)),
)
_mlp_sum = _mlp_call(
    _mlp_sum_body,
    jax.ShapeDtypeStruct((1, D), jnp.float32),
    pl.BlockSpec((1, D), lambda i: (0, 0)),
)


@jax.jit
def kernel(feats, edge_index, W1, b1, W2, b2, eps):
    src = edge_index[0].astype(jnp.int32)
    dst = edge_index[1].astype(jnp.int32)
    pad = E_PAD - N_EDGES
    src = jnp.concatenate([src, jnp.zeros((pad,), jnp.int32)]).reshape(NC * NS, CPW, CHUNK)
    dst = jnp.concatenate([dst, jnp.full((pad,), N_NODES, jnp.int32)]).reshape(
        NC * NS, CPW, CHUNK
    )

    h = feats
    for i in range(NUM_LAYERS):
        agg = _sc_agg(src, dst, h)
        a0 = agg[0, :N_NODES]
        a1 = agg[1, :N_NODES]
        eps_i = eps[i].reshape(1, 1)
        args = (eps_i, h, a0, a1, W1[i], b1[i].reshape(1, D), W2[i], b2[i].reshape(1, D))
        if i < NUM_LAYERS - 1:
            h = _mlp(*args)
        else:
            return _mlp_sum(*args)
